# baseline (speedup 1.0000x reference)
import jax
import jax.numpy as jnp
from jax.experimental import pallas as pl

N = 10000
E = 160000
B = 16
HID = 256
EH = 64
NPROC = 192


def _ln(h):
    mu = h.mean(-1, keepdims=True)
    v = ((h - mu) ** 2).mean(-1, keepdims=True)
    return (h - mu) / jnp.sqrt(v + 1e-5)


def _lin(h, p):
    return h @ p["W"] + p["b"]


def _mk_head_pallas(xp, pm):
    def body(xp_ref, w0, b0, w1, b1, w2, b2, w3, b3, out_ref):
        z = xp_ref[...] @ w0[...] + b0[...]
        mu = z.mean(-1, keepdims=True)
        v = ((z - mu) ** 2).mean(-1, keepdims=True)
        z = jnp.maximum((z - mu) / jnp.sqrt(v + 1e-5), 0.0)
        z = z @ w1[...] + b1[...]
        mu = z.mean(-1, keepdims=True)
        v = ((z - mu) ** 2).mean(-1, keepdims=True)
        z = jnp.maximum((z - mu) / jnp.sqrt(v + 1e-5), 0.0)
        z = jnp.maximum(z @ w2[...] + b2[...], 0.0)
        out_ref[...] = z @ w3[...] + b3[...]

    args = [xp]
    for p in pm:
        args += [p["W"], p["b"].reshape(1, -1)]
    return pl.pallas_call(
        body,
        out_shape=jax.ShapeDtypeStruct((B, 1), jnp.float32),
    )(*args)


def kernel(x, edge_index, edge_attr, batch, params):
    src, dst = edge_index[0], edge_index[1]
    h = jax.nn.relu(_ln(_lin(x, params["ne"])))
    e = jax.nn.relu(_lin(edge_attr, params["ee"]))
    deg = jnp.zeros((N,), jnp.float32).at[dst].add(1.0)
    loop_e = jax.ops.segment_sum(e, dst, num_segments=N) / jnp.maximum(deg, 1.0)[:, None]
    ar = jnp.arange(N, dtype=src.dtype)
    src2 = jnp.concatenate([src, ar])
    dst2 = jnp.concatenate([dst, ar])
    e2 = jnp.concatenate([e, loop_e], axis=0)

    def gat(h, p, H, C, concat):
        xl = (h @ p["Wl"] + p["bl"]).reshape(N, H, C)
        xr = (h @ p["Wr"] + p["br"]).reshape(N, H, C)
        m = xl[src2] + xr[dst2] + (e2 @ p["We"]).reshape(-1, H, C)
        a = jax.nn.leaky_relu(m, 0.2)
        alpha = jnp.einsum("ehc,hc->eh", a, p["att"])
        amax = jax.ops.segment_max(alpha, dst2, num_segments=N)
        amax = jnp.where(jnp.isfinite(amax), amax, 0.0)
        ex = jnp.exp(alpha - amax[dst2])
        den = jax.ops.segment_sum(ex, dst2, num_segments=N)
        al = ex / (den[dst2] + 1e-16)
        out = jax.ops.segment_sum(xl[src2] * al[..., None], dst2, num_segments=N)
        out = out.reshape(N, H * C) if concat else out.mean(axis=1)
        return out + p["b"]

    for i in range(4):
        p = params["gat"][i]
        hn = gat(h, p, 8, 32, True) if i < 3 else gat(h, p, 1, HID, False)
        hn = jax.nn.relu(_ln(hn))
        h = h + hn if i > 0 else hn

    pp = params["proc"]
    z = jax.nn.relu(_ln(_lin(h, pp[0])))
    z = jax.nn.relu(_lin(z, pp[1]))
    proc = _lin(z, pp[2])
    ps = params["st"]
    z = jax.nn.relu(_ln(_lin(h, ps[0])))
    z = jax.nn.relu(_lin(z, ps[1]))
    st = _lin(z, ps[2])
    pe = params["et"]
    z = jax.nn.relu(_ln(_lin(h, pe[0])))
    z = jax.nn.relu(_lin(z, pe[1]))
    et = _lin(z, pe[2])
    cnt = jnp.zeros((B,), jnp.float32).at[batch].add(1.0)
    xsum = jax.ops.segment_sum(h, batch, num_segments=B)
    xmean = xsum / jnp.maximum(cnt, 1.0)[:, None]
    xmax = jax.ops.segment_max(h, batch, num_segments=B)
    xmax = jnp.where(jnp.isfinite(xmax), xmax, 0.0)
    w = jax.nn.softmax(params["pw"])
    xp = jnp.concatenate([xmean * w[0], xmax * w[1], xsum * w[2]], axis=1)
    mk = _mk_head_pallas(xp, params["mk"])
    return (proc, st, et, mk)


# speedup vs baseline: 12.1333x; 12.1333x over previous
import functools

import jax
import jax.numpy as jnp
from jax import lax
from jax.experimental import pallas as pl
from jax.experimental.pallas import tpu as pltpu
from jax.experimental.pallas import tpu_sc as plsc

N = 10000
E = 160000
B = 16
HID = 256
EH = 64
NPROC = 192

NC, NS = 2, 16
NW = NC * NS
CHUNK = 128
EPAD = 163840
E2 = E + N
E2PAD = 172032
NPAD = 10112

_MESH = dict(core_axis_name="c", subcore_axis_name="s", num_cores=NC,
             num_subcores=NS)


def _sc_gather(d, m_pad, name):
    per_w = m_pad // NW
    n_iter = per_w // CHUNK
    mesh = plsc.VectorSubcoreMesh(**_MESH)

    @functools.partial(
        pl.kernel,
        out_type=jax.ShapeDtypeStruct((m_pad, d), jnp.float32),
        mesh=mesh,
        scratch_types=[
            pltpu.VMEM((CHUNK,), jnp.int32),
            pltpu.VMEM((CHUNK, d), jnp.float32),
            pltpu.SemaphoreType.DMA,
        ],
        name=name,
    )
    def k(table_hbm, idx_hbm, out_hbm, idx_v, rows_v, sem):
        wid = lax.axis_index("s") * NC + lax.axis_index("c")
        base = wid * per_w

        def body(i, carry):
            off = base + i * CHUNK
            pltpu.sync_copy(idx_hbm.at[pl.ds(off, CHUNK)], idx_v)
            pltpu.async_copy(table_hbm.at[idx_v], rows_v, sem).wait()
            pltpu.sync_copy(rows_v, out_hbm.at[pl.ds(off, CHUNK)])
            return carry

        lax.fori_loop(0, n_iter, body, 0)

    return k


def _sc_scatter(n_rows, d, m_pad, name):
    per_w = m_pad // NW
    n_iter = per_w // CHUNK
    rpt = n_rows // NS
    mesh = plsc.VectorSubcoreMesh(**_MESH)

    @functools.partial(
        pl.kernel,
        out_type=jax.ShapeDtypeStruct((NC, n_rows, d), jnp.float32),
        mesh=mesh,
        scratch_types=[
            pltpu.VMEM((CHUNK,), jnp.int32),
            pltpu.VMEM((CHUNK, d), jnp.float32),
            pltpu.VMEM_SHARED((n_rows, d), jnp.float32),
            pltpu.SemaphoreType.DMA,
        ],
        name=name,
    )
    def k(vals_hbm, idx_hbm, zeros_hbm, out_hbm, idx_v, rows_v, acc_sh, sem):
        c = lax.axis_index("c")
        s = lax.axis_index("s")
        wid = s * NC + c
        r0 = s * rpt
        pltpu.sync_copy(zeros_hbm.at[pl.ds(r0, rpt)], acc_sh.at[pl.ds(r0, rpt)])
        plsc.subcore_barrier()
        base = wid * per_w

        def body(i, carry):
            off = base + i * CHUNK
            pltpu.sync_copy(idx_hbm.at[pl.ds(off, CHUNK)], idx_v)
            pltpu.sync_copy(vals_hbm.at[pl.ds(off, CHUNK)], rows_v)
            pltpu.sync_copy(rows_v, acc_sh.at[idx_v], add=True)
            return carry

        lax.fori_loop(0, n_iter, body, 0)
        plsc.subcore_barrier()
        pltpu.sync_copy(acc_sh.at[pl.ds(r0, rpt)], out_hbm.at[c, pl.ds(r0, rpt)])

    return k


def _ln_relu(z):
    mu = jnp.mean(z, axis=-1, keepdims=True)
    v = jnp.mean((z - mu) ** 2, axis=-1, keepdims=True)
    return jnp.maximum((z - mu) / jnp.sqrt(v + 1e-5), 0.0)


def _dot(a, b):
    return jnp.dot(a, b, preferred_element_type=jnp.float32)


def _full(shape):
    return pl.BlockSpec(shape, lambda i: tuple(0 for _ in shape))


def _encode_node(x8, w8, b):
    blk = 2000

    def body(x_ref, w_ref, b_ref, o_ref):
        o_ref[...] = _ln_relu(_dot(x_ref[...], w_ref[...]) + b_ref[...])

    return pl.pallas_call(
        body,
        grid=(N // blk,),
        in_specs=[pl.BlockSpec((blk, 8), lambda i: (i, 0)),
                  _full((8, HID)), _full((1, HID))],
        out_specs=pl.BlockSpec((blk, HID), lambda i: (i, 0)),
        out_shape=jax.ShapeDtypeStruct((N, HID), jnp.float32),
    )(x8, w8, b)


def _encode_edge(ea_pad, w128, b128):
    blk = 2048

    def body(a_ref, w_ref, b_ref, o_ref):
        pid = pl.program_id(0)
        rows = lax.broadcasted_iota(jnp.int32, (blk, 1), 0) + pid * blk
        valid = rows < E
        ez = jnp.maximum(a_ref[...] * w_ref[...] + b_ref[...], 0.0)
        colio = lax.broadcasted_iota(jnp.int32, (blk, 128), 1)
        res = ez + jnp.where(colio == 64, 1.0, 0.0)
        o_ref[...] = jnp.where(valid, res, 0.0)

    return pl.pallas_call(
        body,
        grid=(EPAD // blk,),
        in_specs=[pl.BlockSpec((blk, 1), lambda i: (i, 0)),
                  _full((1, 128)), _full((1, 128))],
        out_specs=pl.BlockSpec((blk, 128), lambda i: (i, 0)),
        out_shape=jax.ShapeDtypeStruct((EPAD, 128), jnp.float32),
    )(ea_pad, w128, b128)


def _loope_div(parts, sel_e, sel_deg):
    blk = 2000

    def body(p_ref, se_ref, sd_ref, o_ref):
        s = p_ref[0] + p_ref[1]
        esum = _dot(s, se_ref[...])
        deg = _dot(s, sd_ref[...])
        o_ref[...] = esum / jnp.maximum(deg, 1.0)

    return pl.pallas_call(
        body,
        grid=(N // blk,),
        in_specs=[pl.BlockSpec((NC, blk, 128), lambda i: (0, i, 0)),
                  _full((128, EH)), _full((128, EH))],
        out_specs=pl.BlockSpec((blk, EH), lambda i: (i, 0)),
        out_shape=jax.ShapeDtypeStruct((N, EH), jnp.float32),
    )(parts, sel_e, sel_deg)


def _proj(h, loop_e, wl, bl, wr, br, we, attbd):
    blk = 2000

    def body(h_ref, le_ref, wl_ref, bl_ref, wr_ref, br_ref, we_ref, at_ref,
             xl_ref, xr_ref):
        h_b = h_ref[...]
        xl = _dot(h_b, wl_ref[...]) + bl_ref[...]
        xr = _dot(h_b, wr_ref[...]) + br_ref[...]
        ul = _dot(le_ref[...], we_ref[...])
        m = xl + xr + ul
        a = jnp.where(m > 0, m, 0.2 * m)
        c128 = _dot(a, at_ref[...])
        xl_ref[...] = xl
        xr_ref[...] = jnp.concatenate([xr, c128], axis=1)

    return pl.pallas_call(
        body,
        grid=(N // blk,),
        in_specs=[pl.BlockSpec((blk, HID), lambda i: (i, 0)),
                  pl.BlockSpec((blk, EH), lambda i: (i, 0)),
                  _full((HID, HID)), _full((1, HID)),
                  _full((HID, HID)), _full((1, HID)),
                  _full((EH, HID)), _full((HID, 128))],
        out_specs=[pl.BlockSpec((blk, HID), lambda i: (i, 0)),
                   pl.BlockSpec((blk, HID + 128), lambda i: (i, 0))],
        out_shape=[jax.ShapeDtypeStruct((N, HID), jnp.float32),
                   jax.ShapeDtypeStruct((N, HID + 128), jnp.float32)],
    )(h, loop_e, wl, bl, wr, br, we, attbd)


def _edge_mm(e2p, we):
    blk = 2048

    def body(e_ref, w_ref, o_ref):
        o_ref[...] = _dot(e_ref[...], w_ref[...])

    return pl.pallas_call(
        body,
        grid=(E2PAD // blk,),
        in_specs=[pl.BlockSpec((blk, EH), lambda i: (i, 0)),
                  _full((EH, HID))],
        out_specs=pl.BlockSpec((blk, HID), lambda i: (i, 0)),
        out_shape=jax.ShapeDtypeStruct((E2PAD, HID), jnp.float32),
    )(e2p, we)


def _alpha_ex(xlg, xrg, u, attbd, nheads):
    blk = 2048

    def body(xl_ref, xr_ref, u_ref, at_ref, ex_ref):
        pid = pl.program_id(0)
        xr384 = xr_ref[...]
        m = xl_ref[...] + xr384[:, :HID] + u_ref[...]
        a = jnp.where(m > 0, m, 0.2 * m)
        alpha = _dot(a, at_ref[...])
        ex = jnp.exp(alpha - xr384[:, HID:])
        rows = lax.broadcasted_iota(jnp.int32, (blk, 1), 0) + pid * blk
        cols = lax.broadcasted_iota(jnp.int32, (blk, 128), 1)
        ex_ref[...] = jnp.where((rows < E2) & (cols < nheads), ex, 0.0)

    return pl.pallas_call(
        body,
        grid=(E2PAD // blk,),
        in_specs=[pl.BlockSpec((blk, HID), lambda i: (i, 0)),
                  pl.BlockSpec((blk, HID + 128), lambda i: (i, 0)),
                  pl.BlockSpec((blk, HID), lambda i: (i, 0)),
                  _full((HID, 128))],
        out_specs=pl.BlockSpec((blk, 128), lambda i: (i, 0)),
        out_shape=jax.ShapeDtypeStruct((E2PAD, 128), jnp.float32),
    )(xlg, xrg, u, attbd)


def _msg(xlg, ex, expand):
    blk = 2048

    def body(xl_ref, ex_ref, exp_ref, m0_ref, m1_ref):
        alx = _dot(ex_ref[...], exp_ref[...])
        prod = xl_ref[...] * alx
        m0_ref[...] = prod[:, :128]
        m1_ref[...] = prod[:, 128:]

    return pl.pallas_call(
        body,
        grid=(E2PAD // blk,),
        in_specs=[pl.BlockSpec((blk, HID), lambda i: (i, 0)),
                  pl.BlockSpec((blk, 128), lambda i: (i, 0)),
                  _full((128, HID))],
        out_specs=[pl.BlockSpec((blk, 128), lambda i: (i, 0)),
                   pl.BlockSpec((blk, 128), lambda i: (i, 0))],
        out_shape=[jax.ShapeDtypeStruct((E2PAD, 128), jnp.float32),
                   jax.ShapeDtypeStruct((E2PAD, 128), jnp.float32)],
    )(xlg, ex, expand)


def _layer_out(p0, p1, den, expand, bias, hprev, residual):
    blk = 2000

    def body(p0_ref, p1_ref, dn_ref, exp_ref, b_ref, h_ref, o_ref):
        s0 = p0_ref[0] + p0_ref[1]
        s1 = p1_ref[0] + p1_ref[1]
        den = dn_ref[0] + dn_ref[1]
        denx = _dot(den, exp_ref[...]) + 1e-16
        raw = jnp.concatenate([s0, s1], axis=1)
        out = raw / denx + b_ref[...]
        hn = _ln_relu(out)
        o_ref[...] = h_ref[...] + hn if residual else hn

    return pl.pallas_call(
        body,
        grid=(N // blk,),
        in_specs=[pl.BlockSpec((NC, blk, 128), lambda i: (0, i, 0)),
                  pl.BlockSpec((NC, blk, 128), lambda i: (0, i, 0)),
                  pl.BlockSpec((NC, blk, 128), lambda i: (0, i, 0)),
                  _full((128, HID)),
                  _full((1, HID)),
                  pl.BlockSpec((blk, HID), lambda i: (i, 0))],
        out_specs=pl.BlockSpec((blk, HID), lambda i: (i, 0)),
        out_shape=jax.ShapeDtypeStruct((N, HID), jnp.float32),
    )(p0, p1, den, expand, bias, hprev)


def _head(h, ps, dims):
    blk = 2000
    d0, d1, d2 = dims

    def body(h_ref, w0, b0, w1, b1, w2, b2, o_ref):
        z = _ln_relu(_dot(h_ref[...], w0[...]) + b0[...])
        z = jnp.maximum(_dot(z, w1[...]) + b1[...], 0.0)
        o_ref[...] = _dot(z, w2[...]) + b2[...]

    return pl.pallas_call(
        body,
        grid=(N // blk,),
        in_specs=[pl.BlockSpec((blk, HID), lambda i: (i, 0)),
                  _full((HID, d0)), _full((1, d0)),
                  _full((d0, d1)), _full((1, d1)),
                  _full((d1, d2)), _full((1, d2))],
        out_specs=pl.BlockSpec((blk, d2), lambda i: (i, 0)),
        out_shape=jax.ShapeDtypeStruct((N, d2), jnp.float32),
    )(h, ps[0]["W"], ps[0]["b"].reshape(1, -1),
      ps[1]["W"], ps[1]["b"].reshape(1, -1),
      ps[2]["W"], ps[2]["b"].reshape(1, -1))


def _pool(h, batch_col):
    blk = 2000

    def body(h_ref, b_ref, xsum_ref, xmax_ref, cnt_ref):
        @pl.when(pl.program_id(0) == 0)
        def _init():
            xsum_ref[...] = jnp.zeros_like(xsum_ref)
            xmax_ref[...] = jnp.full_like(xmax_ref, -jnp.inf)
            cnt_ref[...] = jnp.zeros_like(cnt_ref)

        hb = h_ref[...]
        bb = b_ref[...]
        biota = lax.broadcasted_iota(jnp.int32, (blk, B), 1)
        oh = (bb == biota).astype(jnp.float32)
        xsum_ref[...] += lax.dot_general(
            oh, hb, (((0,), (0,)), ((), ())),
            preferred_element_type=jnp.float32)
        cnt_ref[...] += lax.dot_general(
            oh, jnp.ones((blk, 128), jnp.float32), (((0,), (0,)), ((), ())),
            preferred_element_type=jnp.float32)
        for bseg in range(B):
            mb = bb == bseg
            mx = jnp.max(jnp.where(mb, hb, -jnp.inf), axis=0, keepdims=True)
            xmax_ref[bseg:bseg + 1, :] = jnp.maximum(
                xmax_ref[bseg:bseg + 1, :], mx)

    return pl.pallas_call(
        body,
        grid=(N // blk,),
        in_specs=[pl.BlockSpec((blk, HID), lambda i: (i, 0)),
                  pl.BlockSpec((blk, 1), lambda i: (i, 0))],
        out_specs=[_full((B, HID)), _full((B, HID)), _full((B, 128))],
        out_shape=[jax.ShapeDtypeStruct((B, HID), jnp.float32),
                   jax.ShapeDtypeStruct((B, HID), jnp.float32),
                   jax.ShapeDtypeStruct((B, 128), jnp.float32)],
    )(h, batch_col)


def _mk_head(xsum, xmax, cnt, pw_pad, pm):
    def body(xs_ref, xm_ref, cn_ref, pw_ref, w0, b0, w1, b1, w2, b2, w3, b3,
             o_ref):
        pw = pw_ref[...]
        mxw = jnp.max(pw, axis=-1, keepdims=True)
        ew = jnp.exp(pw - mxw)
        w = ew / jnp.sum(ew, axis=-1, keepdims=True)
        w0_, w1_, w2_ = w[:, 0:1], w[:, 1:2], w[:, 2:3]
        cnt1 = jnp.maximum(cn_ref[:, 0:1], 1.0)
        xsum = xs_ref[...]
        xmean = xsum / cnt1
        xmax = xm_ref[...]
        xmax = jnp.where(xmax > -3e38, xmax, 0.0)
        xp = jnp.concatenate([xmean * w0_, xmax * w1_, xsum * w2_], axis=1)
        z = _ln_relu(_dot(xp, w0[...]) + b0[...])
        z = _ln_relu(_dot(z, w1[...]) + b1[...])
        z = jnp.maximum(_dot(z, w2[...]) + b2[...], 0.0)
        o_ref[...] = _dot(z, w3[...]) + b3[...]

    args = [xsum, xmax, cnt, pw_pad]
    for p in pm:
        args += [p["W"], p["b"].reshape(1, -1)]
    return pl.pallas_call(
        body,
        out_shape=jax.ShapeDtypeStruct((B, 1), jnp.float32),
    )(*args)


def kernel(x, edge_index, edge_attr, batch, params):
    f32 = jnp.float32
    src, dst = edge_index[0], edge_index[1]
    ar = jnp.arange(N, dtype=jnp.int32)
    zpad2 = jnp.zeros((E2PAD - E2,), jnp.int32)
    src2p = jnp.concatenate([src, ar, zpad2])
    dst2p = jnp.concatenate([dst, ar, zpad2])
    dstp = jnp.concatenate([dst, jnp.zeros((EPAD - E,), jnp.int32)])

    x8 = jnp.pad(x, ((0, 0), (0, 5)))
    w8 = jnp.pad(params["ne"]["W"], ((0, 5), (0, 0)))
    h = _encode_node(x8, w8, params["ne"]["b"].reshape(1, -1))
    ea_pad = jnp.pad(edge_attr, ((0, EPAD - E), (0, 0)))
    w128 = jnp.pad(params["ee"]["W"], ((0, 0), (0, 64)))
    b128 = jnp.pad(params["ee"]["b"].reshape(1, -1), ((0, 0), (0, 64)))
    vals128 = _encode_edge(ea_pad, w128, b128)

    zeros_n128 = jnp.zeros((NPAD, 128), f32)
    le_parts = _sc_scatter(NPAD, 128, EPAD, "scat_loope")(vals128, dstp,
                                                          zeros_n128)
    colio = jnp.arange(128)
    sel_e = (colio[:, None] == jnp.arange(EH)[None, :]).astype(f32)
    sel_deg = (colio[:, None] == 64).astype(f32) * jnp.ones((1, EH), f32)
    loop_e = _loope_div(le_parts, sel_e, sel_deg)

    e2p = jnp.concatenate(
        [lax.slice(vals128, (0, 0), (E, EH)), loop_e,
         jnp.zeros((E2PAD - E2, EH), f32)], axis=0)

    gat_h = _sc_gather(HID, E2PAD, "gather_hid")
    gat_384 = _sc_gather(HID + 128, E2PAD, "gather_384")
    scat_den = _sc_scatter(NPAD, 128, E2PAD, "scat_den")
    scat_msg = _sc_scatter(NPAD, 128, E2PAD, "scat_msg")

    idxh = jnp.arange(HID)
    for i in range(4):
        p = params["gat"][i]
        nh = 8 if i < 3 else 1
        ch = HID // nh
        attbd = jnp.zeros((HID, 128), f32).at[idxh, idxh // ch].set(
            p["att"].reshape(-1))
        expand = (jnp.arange(128)[:, None] == (idxh[None, :] // ch)).astype(f32)
        xl, xr384 = _proj(h, loop_e, p["Wl"], p["bl"].reshape(1, -1),
                          p["Wr"], p["br"].reshape(1, -1), p["We"], attbd)
        u = _edge_mm(e2p, p["We"])
        xlg = gat_h(xl, src2p)
        xrg = gat_384(xr384, dst2p)
        ex = _alpha_ex(xlg, xrg, u, attbd, nh)
        den_parts = scat_den(ex, dst2p, zeros_n128)
        msg0, msg1 = _msg(xlg, ex, expand)
        p0 = scat_msg(msg0, dst2p, zeros_n128)
        p1 = scat_msg(msg1, dst2p, zeros_n128)
        h = _layer_out(p0, p1, den_parts, expand, p["b"].reshape(1, -1), h,
                       residual=(i > 0))

    proc = _head(h, params["proc"], (HID, HID // 2, NPROC))
    st = _head(h, params["st"], (HID // 2, HID // 4, 1))
    et = _head(h, params["et"], (HID // 2, HID // 4, 1))

    xsum, xmax, cnt = _pool(h, batch.reshape(-1, 1))
    pw_pad = jnp.concatenate(
        [params["pw"], jnp.full((125,), -jnp.inf, f32)]).reshape(1, 128)
    mk = _mk_head(xsum, xmax, cnt, pw_pad, params["mk"])
    return (proc, st, et, mk)


# speedup vs baseline: 12.6423x; 1.0420x over previous
import functools

import jax
import jax.numpy as jnp
from jax import lax
from jax.experimental import pallas as pl
from jax.experimental.pallas import tpu as pltpu
from jax.experimental.pallas import tpu_sc as plsc

N = 10000
E = 160000
B = 16
HID = 256
EH = 64
NPROC = 192

NC, NS = 2, 16
NW = NC * NS
CHUNK = 128
EPAD = 163840
E2 = E + N
E2PAD = 172032
NPAD = 10112

_MESH = dict(core_axis_name="c", subcore_axis_name="s", num_cores=NC,
             num_subcores=NS)


def _sc_gather(d, m_pad, name):
    per_w = m_pad // NW
    n_iter = per_w // CHUNK
    mesh = plsc.VectorSubcoreMesh(**_MESH)

    @functools.partial(
        pl.kernel,
        out_type=jax.ShapeDtypeStruct((m_pad, d), jnp.float32),
        mesh=mesh,
        scratch_types=[
            pltpu.VMEM((per_w,), jnp.int32),
            [pltpu.VMEM((CHUNK, d), jnp.float32) for _ in range(2)],
            [pltpu.SemaphoreType.DMA for _ in range(2)],
            [pltpu.SemaphoreType.DMA for _ in range(2)],
        ],
        name=name,
    )
    def k(table_hbm, idx_hbm, out_hbm, idx_all, rows, sg, sw):
        wid = lax.axis_index("s") * NC + lax.axis_index("c")
        base = wid * per_w
        pltpu.sync_copy(idx_hbm.at[pl.ds(base, per_w)], idx_all)
        for b in (0, 1):
            pltpu.async_copy(
                table_hbm.at[idx_all.at[pl.ds(b * CHUNK, CHUNK)]], rows[b],
                sg[b])

        def body(i2, carry):
            for b in (0, 1):
                i = i2 * 2 + b
                off = base + i * CHUNK
                pltpu.make_async_copy(
                    out_hbm.at[pl.ds(off, CHUNK)], rows[b], sg[b]).wait()
                pltpu.async_copy(rows[b], out_hbm.at[pl.ds(off, CHUNK)], sw[b])
            for b in (0, 1):
                i = i2 * 2 + b + 2

                @pl.when(i < n_iter)
                def _():
                    off = base + i * CHUNK
                    pltpu.make_async_copy(
                        rows[b], out_hbm.at[pl.ds(off, CHUNK)], sw[b]).wait()
                    pltpu.async_copy(
                        table_hbm.at[idx_all.at[pl.ds(i * CHUNK, CHUNK)]],
                        rows[b], sg[b])
            return carry

        lax.fori_loop(0, n_iter // 2, body, 0)
        for b in (0, 1):
            off = base + (n_iter - 2 + b) * CHUNK
            pltpu.make_async_copy(
                rows[b], out_hbm.at[pl.ds(off, CHUNK)], sw[b]).wait()

    return k


def _sc_scatter(n_rows, d, m_pad, name):
    per_w = m_pad // NW
    n_iter = per_w // CHUNK
    rpt = n_rows // NS
    mesh = plsc.VectorSubcoreMesh(**_MESH)

    @functools.partial(
        pl.kernel,
        out_type=jax.ShapeDtypeStruct((NC, n_rows, d), jnp.float32),
        mesh=mesh,
        scratch_types=[
            [pltpu.VMEM((CHUNK,), jnp.int32) for _ in range(2)],
            [pltpu.VMEM((CHUNK, d), jnp.float32) for _ in range(2)],
            [pltpu.SemaphoreType.DMA for _ in range(2)],
            [pltpu.SemaphoreType.DMA for _ in range(2)],
            pltpu.VMEM_SHARED((n_rows, d), jnp.float32),
        ],
        name=name,
    )
    def k(vals_hbm, idx_hbm, zeros_hbm, out_hbm, idx_v, rows, sv, sa, acc_sh):
        c = lax.axis_index("c")
        s = lax.axis_index("s")
        wid = s * NC + c
        r0 = s * rpt
        pltpu.sync_copy(zeros_hbm.at[pl.ds(r0, rpt)], acc_sh.at[pl.ds(r0, rpt)])
        plsc.subcore_barrier()
        base = wid * per_w
        for b in (0, 1):
            off = base + b * CHUNK
            pltpu.sync_copy(idx_hbm.at[pl.ds(off, CHUNK)], idx_v[b])
            pltpu.async_copy(vals_hbm.at[pl.ds(off, CHUNK)], rows[b], sv[b])

        def body(i2, carry):
            for b in (0, 1):
                i = i2 * 2 + b
                off = base + i * CHUNK
                pltpu.make_async_copy(
                    vals_hbm.at[pl.ds(off, CHUNK)], rows[b], sv[b]).wait()
                pltpu.async_copy(rows[b], acc_sh.at[idx_v[b]], sa[b], add=True)
            for b in (0, 1):
                i = i2 * 2 + b + 2

                @pl.when(i < n_iter)
                def _():
                    off = base + i * CHUNK
                    pltpu.make_async_copy(
                        rows[b], acc_sh.at[idx_v[b]], sa[b]).wait()
                    pltpu.sync_copy(idx_hbm.at[pl.ds(off, CHUNK)], idx_v[b])
                    pltpu.async_copy(vals_hbm.at[pl.ds(off, CHUNK)], rows[b],
                                     sv[b])
            return carry

        lax.fori_loop(0, n_iter // 2, body, 0)
        for b in (0, 1):
            pltpu.make_async_copy(rows[b], acc_sh.at[idx_v[b]], sa[b]).wait()
        plsc.subcore_barrier()
        pltpu.sync_copy(acc_sh.at[pl.ds(r0, rpt)], out_hbm.at[c, pl.ds(r0, rpt)])

    return k


def _ln_relu(z):
    mu = jnp.mean(z, axis=-1, keepdims=True)
    v = jnp.mean((z - mu) ** 2, axis=-1, keepdims=True)
    return jnp.maximum((z - mu) / jnp.sqrt(v + 1e-5), 0.0)


def _dot(a, b):
    return jnp.dot(a, b, preferred_element_type=jnp.float32)


def _dot_hi(a, b):
    return jnp.dot(a, b, preferred_element_type=jnp.float32,
                   precision=lax.Precision.HIGHEST)


def _full(shape):
    return pl.BlockSpec(shape, lambda i: tuple(0 for _ in shape))


def _encode_node(x8, w8, b):
    blk = 2000

    def body(x_ref, w_ref, b_ref, o_ref):
        o_ref[...] = _ln_relu(_dot(x_ref[...], w_ref[...]) + b_ref[...])

    return pl.pallas_call(
        body,
        grid=(N // blk,),
        in_specs=[pl.BlockSpec((blk, 8), lambda i: (i, 0)),
                  _full((8, HID)), _full((1, HID))],
        out_specs=pl.BlockSpec((blk, HID), lambda i: (i, 0)),
        out_shape=jax.ShapeDtypeStruct((N, HID), jnp.float32),
    )(x8, w8, b)


def _encode_edge(ea_pad, w128, b128):
    blk = 2048

    def body(a_ref, w_ref, b_ref, o_ref):
        pid = pl.program_id(0)
        rows = lax.broadcasted_iota(jnp.int32, (blk, 1), 0) + pid * blk
        valid = rows < E
        ez = jnp.maximum(a_ref[...] * w_ref[...] + b_ref[...], 0.0)
        colio = lax.broadcasted_iota(jnp.int32, (blk, 128), 1)
        res = ez + jnp.where(colio == 64, 1.0, 0.0)
        o_ref[...] = jnp.where(valid, res, 0.0)

    return pl.pallas_call(
        body,
        grid=(EPAD // blk,),
        in_specs=[pl.BlockSpec((blk, 1), lambda i: (i, 0)),
                  _full((1, 128)), _full((1, 128))],
        out_specs=pl.BlockSpec((blk, 128), lambda i: (i, 0)),
        out_shape=jax.ShapeDtypeStruct((EPAD, 128), jnp.float32),
    )(ea_pad, w128, b128)


def _loope_div(parts, sel_e, sel_deg):
    blk = 2000

    def body(p_ref, se_ref, sd_ref, o_ref):
        s = p_ref[0] + p_ref[1]
        esum = _dot_hi(s, se_ref[...])
        deg = _dot_hi(s, sd_ref[...])
        o_ref[...] = esum / jnp.maximum(deg, 1.0)

    return pl.pallas_call(
        body,
        grid=(N // blk,),
        in_specs=[pl.BlockSpec((NC, blk, 128), lambda i: (0, i, 0)),
                  _full((128, EH)), _full((128, EH))],
        out_specs=pl.BlockSpec((blk, EH), lambda i: (i, 0)),
        out_shape=jax.ShapeDtypeStruct((N, EH), jnp.float32),
    )(parts, sel_e, sel_deg)


def _proj(h, loop_e, wl, bl, wr, br, we, attbd):
    blk = 2000

    def body(h_ref, le_ref, wl_ref, bl_ref, wr_ref, br_ref, we_ref, at_ref,
             xl_ref, xr_ref):
        h_b = h_ref[...]
        xl = _dot(h_b, wl_ref[...]) + bl_ref[...]
        xr = _dot(h_b, wr_ref[...]) + br_ref[...]
        ul = _dot(le_ref[...], we_ref[...])
        m = xl + xr + ul
        a = jnp.where(m > 0, m, 0.2 * m)
        c128 = _dot_hi(a, at_ref[...])
        xl_ref[...] = xl
        xr_ref[...] = jnp.concatenate([xr, c128], axis=1)

    return pl.pallas_call(
        body,
        grid=(N // blk,),
        in_specs=[pl.BlockSpec((blk, HID), lambda i: (i, 0)),
                  pl.BlockSpec((blk, EH), lambda i: (i, 0)),
                  _full((HID, HID)), _full((1, HID)),
                  _full((HID, HID)), _full((1, HID)),
                  _full((EH, HID)), _full((HID, 128))],
        out_specs=[pl.BlockSpec((blk, HID), lambda i: (i, 0)),
                   pl.BlockSpec((blk, HID + 128), lambda i: (i, 0))],
        out_shape=[jax.ShapeDtypeStruct((N, HID), jnp.float32),
                   jax.ShapeDtypeStruct((N, HID + 128), jnp.float32)],
    )(h, loop_e, wl, bl, wr, br, we, attbd)


def _edge_fused(xlg, xrg, e2p, we, attbd, expand, nheads):
    blk = 2048

    def body(xl_ref, xr_ref, e_ref, we_ref, at_ref, exp_ref,
             ex_ref, m0_ref, m1_ref):
        pid = pl.program_id(0)
        xr384 = xr_ref[...]
        xl = xl_ref[...]
        u = _dot(e_ref[...], we_ref[...])
        m = xl + xr384[:, :HID] + u
        a = jnp.where(m > 0, m, 0.2 * m)
        alpha = _dot_hi(a, at_ref[...])
        ex = jnp.exp(alpha - xr384[:, HID:])
        rows = lax.broadcasted_iota(jnp.int32, (blk, 1), 0) + pid * blk
        cols = lax.broadcasted_iota(jnp.int32, (blk, 128), 1)
        ex = jnp.where((rows < E2) & (cols < nheads), ex, 0.0)
        ex_ref[...] = ex
        alx = _dot_hi(ex, exp_ref[...])
        prod = xl * alx
        m0_ref[...] = prod[:, :128]
        m1_ref[...] = prod[:, 128:]

    return pl.pallas_call(
        body,
        grid=(E2PAD // blk,),
        in_specs=[pl.BlockSpec((blk, HID), lambda i: (i, 0)),
                  pl.BlockSpec((blk, HID + 128), lambda i: (i, 0)),
                  pl.BlockSpec((blk, EH), lambda i: (i, 0)),
                  _full((EH, HID)), _full((HID, 128)), _full((128, HID))],
        out_specs=[pl.BlockSpec((blk, 128), lambda i: (i, 0)),
                   pl.BlockSpec((blk, 128), lambda i: (i, 0)),
                   pl.BlockSpec((blk, 128), lambda i: (i, 0))],
        out_shape=[jax.ShapeDtypeStruct((E2PAD, 128), jnp.float32),
                   jax.ShapeDtypeStruct((E2PAD, 128), jnp.float32),
                   jax.ShapeDtypeStruct((E2PAD, 128), jnp.float32)],
    )(xlg, xrg, e2p, we, attbd, expand)


def _layer_out(p0, p1, den, expand, bias, hprev, residual):
    blk = 2000

    def body(p0_ref, p1_ref, dn_ref, exp_ref, b_ref, h_ref, o_ref):
        s0 = p0_ref[0] + p0_ref[1]
        s1 = p1_ref[0] + p1_ref[1]
        den = dn_ref[0] + dn_ref[1]
        denx = _dot_hi(den, exp_ref[...]) + 1e-16
        raw = jnp.concatenate([s0, s1], axis=1)
        out = raw / denx + b_ref[...]
        hn = _ln_relu(out)
        o_ref[...] = h_ref[...] + hn if residual else hn

    return pl.pallas_call(
        body,
        grid=(N // blk,),
        in_specs=[pl.BlockSpec((NC, blk, 128), lambda i: (0, i, 0)),
                  pl.BlockSpec((NC, blk, 128), lambda i: (0, i, 0)),
                  pl.BlockSpec((NC, blk, 128), lambda i: (0, i, 0)),
                  _full((128, HID)),
                  _full((1, HID)),
                  pl.BlockSpec((blk, HID), lambda i: (i, 0))],
        out_specs=pl.BlockSpec((blk, HID), lambda i: (i, 0)),
        out_shape=jax.ShapeDtypeStruct((N, HID), jnp.float32),
    )(p0, p1, den, expand, bias, hprev)


def _head(h, ps, dims):
    blk = 2000
    d0, d1, d2 = dims

    def body(h_ref, w0, b0, w1, b1, w2, b2, o_ref):
        z = _ln_relu(_dot(h_ref[...], w0[...]) + b0[...])
        z = jnp.maximum(_dot(z, w1[...]) + b1[...], 0.0)
        o_ref[...] = _dot(z, w2[...]) + b2[...]

    return pl.pallas_call(
        body,
        grid=(N // blk,),
        in_specs=[pl.BlockSpec((blk, HID), lambda i: (i, 0)),
                  _full((HID, d0)), _full((1, d0)),
                  _full((d0, d1)), _full((1, d1)),
                  _full((d1, d2)), _full((1, d2))],
        out_specs=pl.BlockSpec((blk, d2), lambda i: (i, 0)),
        out_shape=jax.ShapeDtypeStruct((N, d2), jnp.float32),
    )(h, ps[0]["W"], ps[0]["b"].reshape(1, -1),
      ps[1]["W"], ps[1]["b"].reshape(1, -1),
      ps[2]["W"], ps[2]["b"].reshape(1, -1))


def _pool(h, batch_col):
    blk = 2000

    def body(h_ref, b_ref, xsum_ref, xmax_ref, cnt_ref):
        @pl.when(pl.program_id(0) == 0)
        def _init():
            xsum_ref[...] = jnp.zeros_like(xsum_ref)
            xmax_ref[...] = jnp.full_like(xmax_ref, -jnp.inf)
            cnt_ref[...] = jnp.zeros_like(cnt_ref)

        hb = h_ref[...]
        bb = b_ref[...]
        biota = lax.broadcasted_iota(jnp.int32, (blk, B), 1)
        oh = (bb == biota).astype(jnp.float32)
        xsum_ref[...] += lax.dot_general(
            oh, hb, (((0,), (0,)), ((), ())),
            preferred_element_type=jnp.float32,
            precision=lax.Precision.HIGHEST)
        cnt_ref[...] += lax.dot_general(
            oh, jnp.ones((blk, 128), jnp.float32), (((0,), (0,)), ((), ())),
            preferred_element_type=jnp.float32,
            precision=lax.Precision.HIGHEST)
        for bseg in range(B):
            mb = bb == bseg
            mx = jnp.max(jnp.where(mb, hb, -jnp.inf), axis=0, keepdims=True)
            xmax_ref[bseg:bseg + 1, :] = jnp.maximum(
                xmax_ref[bseg:bseg + 1, :], mx)

    return pl.pallas_call(
        body,
        grid=(N // blk,),
        in_specs=[pl.BlockSpec((blk, HID), lambda i: (i, 0)),
                  pl.BlockSpec((blk, 1), lambda i: (i, 0))],
        out_specs=[_full((B, HID)), _full((B, HID)), _full((B, 128))],
        out_shape=[jax.ShapeDtypeStruct((B, HID), jnp.float32),
                   jax.ShapeDtypeStruct((B, HID), jnp.float32),
                   jax.ShapeDtypeStruct((B, 128), jnp.float32)],
    )(h, batch_col)


def _mk_head(xsum, xmax, cnt, pw_pad, pm):
    def body(xs_ref, xm_ref, cn_ref, pw_ref, w0, b0, w1, b1, w2, b2, w3, b3,
             o_ref):
        pw = pw_ref[...]
        mxw = jnp.max(pw, axis=-1, keepdims=True)
        ew = jnp.exp(pw - mxw)
        w = ew / jnp.sum(ew, axis=-1, keepdims=True)
        w0_, w1_, w2_ = w[:, 0:1], w[:, 1:2], w[:, 2:3]
        cnt1 = jnp.maximum(cn_ref[:, 0:1], 1.0)
        xsum = xs_ref[...]
        xmean = xsum / cnt1
        xmax = xm_ref[...]
        xmax = jnp.where(xmax > -3e38, xmax, 0.0)
        xp = jnp.concatenate([xmean * w0_, xmax * w1_, xsum * w2_], axis=1)
        z = _ln_relu(_dot(xp, w0[...]) + b0[...])
        z = _ln_relu(_dot(z, w1[...]) + b1[...])
        z = jnp.maximum(_dot(z, w2[...]) + b2[...], 0.0)
        o_ref[...] = _dot(z, w3[...]) + b3[...]

    args = [xsum, xmax, cnt, pw_pad]
    for p in pm:
        args += [p["W"], p["b"].reshape(1, -1)]
    return pl.pallas_call(
        body,
        out_shape=jax.ShapeDtypeStruct((B, 1), jnp.float32),
    )(*args)


def kernel(x, edge_index, edge_attr, batch, params):
    f32 = jnp.float32
    src, dst = edge_index[0], edge_index[1]
    ar = jnp.arange(N, dtype=jnp.int32)
    zpad2 = jnp.zeros((E2PAD - E2,), jnp.int32)
    src2p = jnp.concatenate([src, ar, zpad2])
    dst2p = jnp.concatenate([dst, ar, zpad2])
    dstp = jnp.concatenate([dst, jnp.zeros((EPAD - E,), jnp.int32)])

    x8 = jnp.pad(x, ((0, 0), (0, 5)))
    w8 = jnp.pad(params["ne"]["W"], ((0, 5), (0, 0)))
    h = _encode_node(x8, w8, params["ne"]["b"].reshape(1, -1))
    ea_pad = jnp.pad(edge_attr, ((0, EPAD - E), (0, 0)))
    w128 = jnp.pad(params["ee"]["W"], ((0, 0), (0, 64)))
    b128 = jnp.pad(params["ee"]["b"].reshape(1, -1), ((0, 0), (0, 64)))
    vals128 = _encode_edge(ea_pad, w128, b128)

    zeros_n128 = jnp.zeros((NPAD, 128), f32)
    le_parts = _sc_scatter(NPAD, 128, EPAD, "scat_loope")(vals128, dstp,
                                                          zeros_n128)
    colio = jnp.arange(128)
    sel_e = (colio[:, None] == jnp.arange(EH)[None, :]).astype(f32)
    sel_deg = (colio[:, None] == 64).astype(f32) * jnp.ones((1, EH), f32)
    loop_e = _loope_div(le_parts, sel_e, sel_deg)

    e2p = jnp.concatenate(
        [lax.slice(vals128, (0, 0), (E, EH)), loop_e,
         jnp.zeros((E2PAD - E2, EH), f32)], axis=0)

    gat_h = _sc_gather(HID, E2PAD, "gather_hid")
    gat_384 = _sc_gather(HID + 128, E2PAD, "gather_384")
    scat_den = _sc_scatter(NPAD, 128, E2PAD, "scat_den")
    scat_msg = _sc_scatter(NPAD, 128, E2PAD, "scat_msg")

    idxh = jnp.arange(HID)
    for i in range(4):
        p = params["gat"][i]
        nh = 8 if i < 3 else 1
        ch = HID // nh
        attbd = jnp.zeros((HID, 128), f32).at[idxh, idxh // ch].set(
            p["att"].reshape(-1))
        expand = (jnp.arange(128)[:, None] == (idxh[None, :] // ch)).astype(f32)
        xl, xr384 = _proj(h, loop_e, p["Wl"], p["bl"].reshape(1, -1),
                          p["Wr"], p["br"].reshape(1, -1), p["We"], attbd)
        xlg = gat_h(xl, src2p)
        xrg = gat_384(xr384, dst2p)
        ex, msg0, msg1 = _edge_fused(xlg, xrg, e2p, p["We"], attbd, expand, nh)
        den_parts = scat_den(ex, dst2p, zeros_n128)
        p0 = scat_msg(msg0, dst2p, zeros_n128)
        p1 = scat_msg(msg1, dst2p, zeros_n128)
        h = _layer_out(p0, p1, den_parts, expand, p["b"].reshape(1, -1), h,
                       residual=(i > 0))

    proc = _head(h, params["proc"], (HID, HID // 2, NPROC))
    st = _head(h, params["st"], (HID // 2, HID // 4, 1))
    et = _head(h, params["et"], (HID // 2, HID // 4, 1))

    xsum, xmax, cnt = _pool(h, batch.reshape(-1, 1))
    pw_pad = jnp.concatenate(
        [params["pw"], jnp.full((125,), -jnp.inf, f32)]).reshape(1, 128)
    mk = _mk_head(xsum, xmax, cnt, pw_pad, params["mk"])
    return (proc, st, et, mk)


# speedup vs baseline: 12.7315x; 1.0071x over previous
import functools

import jax
import jax.numpy as jnp
from jax import lax
from jax.experimental import pallas as pl
from jax.experimental.pallas import tpu as pltpu
from jax.experimental.pallas import tpu_sc as plsc

N = 10000
E = 160000
B = 16
HID = 256
EH = 64
NPROC = 192

NC, NS = 2, 16
NW = NC * NS
CHUNK = 128
EPAD = 163840
E2 = E + N
E2PAD = 172032
NPAD = 10112

_MESH = dict(core_axis_name="c", subcore_axis_name="s", num_cores=NC,
             num_subcores=NS)


def _sc_gather(d, m_pad, name, slots=2):
    per_w = m_pad // NW
    n_iter = per_w // CHUNK
    assert n_iter % slots == 0
    mesh = plsc.VectorSubcoreMesh(**_MESH)

    @functools.partial(
        pl.kernel,
        out_type=jax.ShapeDtypeStruct((m_pad, d), jnp.float32),
        mesh=mesh,
        scratch_types=[
            pltpu.VMEM((per_w,), jnp.int32),
            [pltpu.VMEM((CHUNK, d), jnp.float32) for _ in range(slots)],
            [pltpu.SemaphoreType.DMA for _ in range(slots)],
            [pltpu.SemaphoreType.DMA for _ in range(slots)],
        ],
        name=name,
    )
    def k(table_hbm, idx_hbm, out_hbm, idx_all, rows, sg, sw):
        wid = lax.axis_index("s") * NC + lax.axis_index("c")
        base = wid * per_w
        pltpu.sync_copy(idx_hbm.at[pl.ds(base, per_w)], idx_all)
        for b in range(slots):
            pltpu.async_copy(
                table_hbm.at[idx_all.at[pl.ds(b * CHUNK, CHUNK)]], rows[b],
                sg[b])

        def body(i2, carry):
            for b in range(slots):
                i = i2 * slots + b
                off = base + i * CHUNK
                pltpu.make_async_copy(
                    out_hbm.at[pl.ds(off, CHUNK)], rows[b], sg[b]).wait()
                pltpu.async_copy(rows[b], out_hbm.at[pl.ds(off, CHUNK)], sw[b])
            for b in range(slots):
                i = i2 * slots + b + slots

                @pl.when(i < n_iter)
                def _():
                    off = base + i * CHUNK
                    pltpu.make_async_copy(
                        rows[b], out_hbm.at[pl.ds(off, CHUNK)], sw[b]).wait()
                    pltpu.async_copy(
                        table_hbm.at[idx_all.at[pl.ds(i * CHUNK, CHUNK)]],
                        rows[b], sg[b])
            return carry

        lax.fori_loop(0, n_iter // slots, body, 0)
        for b in range(slots):
            off = base + (n_iter - slots + b) * CHUNK
            pltpu.make_async_copy(
                rows[b], out_hbm.at[pl.ds(off, CHUNK)], sw[b]).wait()

    return k


def _sc_scatter(n_rows, d, m_pad, name, slots=3, chunk=CHUNK):
    per_w = m_pad // NW
    n_iter = per_w // chunk
    assert n_iter % slots == 0
    rpt = n_rows // NS
    mesh = plsc.VectorSubcoreMesh(**_MESH)

    @functools.partial(
        pl.kernel,
        out_type=jax.ShapeDtypeStruct((NC, n_rows, d), jnp.float32),
        mesh=mesh,
        scratch_types=[
            [pltpu.VMEM((chunk,), jnp.int32) for _ in range(slots)],
            [pltpu.VMEM((chunk, d), jnp.float32) for _ in range(slots)],
            [pltpu.SemaphoreType.DMA for _ in range(slots)],
            [pltpu.SemaphoreType.DMA for _ in range(slots)],
            pltpu.VMEM_SHARED((n_rows, d), jnp.float32),
        ],
        name=name,
    )
    def k(vals_hbm, idx_hbm, zeros_hbm, out_hbm, idx_v, rows, sv, sa, acc_sh):
        c = lax.axis_index("c")
        s = lax.axis_index("s")
        wid = s * NC + c
        r0 = s * rpt
        pltpu.sync_copy(zeros_hbm.at[pl.ds(r0, rpt)], acc_sh.at[pl.ds(r0, rpt)])
        plsc.subcore_barrier()
        base = wid * per_w
        for b in range(slots):
            off = base + b * chunk
            pltpu.sync_copy(idx_hbm.at[pl.ds(off, chunk)], idx_v[b])
            pltpu.async_copy(vals_hbm.at[pl.ds(off, chunk)], rows[b], sv[b])

        def body(i2, carry):
            for b in range(slots):
                i = i2 * slots + b
                off = base + i * chunk
                pltpu.make_async_copy(
                    vals_hbm.at[pl.ds(off, chunk)], rows[b], sv[b]).wait()
                pltpu.async_copy(rows[b], acc_sh.at[idx_v[b]], sa[b], add=True)
            for b in range(slots):
                i = i2 * slots + b + slots

                @pl.when(i < n_iter)
                def _():
                    off = base + i * chunk
                    pltpu.make_async_copy(
                        rows[b], acc_sh.at[idx_v[b]], sa[b]).wait()
                    pltpu.sync_copy(idx_hbm.at[pl.ds(off, chunk)], idx_v[b])
                    pltpu.async_copy(vals_hbm.at[pl.ds(off, chunk)], rows[b],
                                     sv[b])
            return carry

        lax.fori_loop(0, n_iter // slots, body, 0)
        for b in range(slots):
            pltpu.make_async_copy(rows[b], acc_sh.at[idx_v[b]], sa[b]).wait()
        plsc.subcore_barrier()
        pltpu.sync_copy(acc_sh.at[pl.ds(r0, rpt)], out_hbm.at[c, pl.ds(r0, rpt)])

    return k


def _ln_relu(z):
    mu = jnp.mean(z, axis=-1, keepdims=True)
    v = jnp.mean((z - mu) ** 2, axis=-1, keepdims=True)
    return jnp.maximum((z - mu) / jnp.sqrt(v + 1e-5), 0.0)


def _dot(a, b):
    return jnp.dot(a, b, preferred_element_type=jnp.float32)


def _dot_hi(a, b):
    return jnp.dot(a, b, preferred_element_type=jnp.float32,
                   precision=lax.Precision.HIGHEST)


def _full(shape):
    return pl.BlockSpec(shape, lambda i: tuple(0 for _ in shape))


def _encode_node(x8, w8, b):
    blk = 2000

    def body(x_ref, w_ref, b_ref, o_ref):
        o_ref[...] = _ln_relu(_dot(x_ref[...], w_ref[...]) + b_ref[...])

    return pl.pallas_call(
        body,
        grid=(N // blk,),
        in_specs=[pl.BlockSpec((blk, 8), lambda i: (i, 0)),
                  _full((8, HID)), _full((1, HID))],
        out_specs=pl.BlockSpec((blk, HID), lambda i: (i, 0)),
        out_shape=jax.ShapeDtypeStruct((N, HID), jnp.float32),
    )(x8, w8, b)


def _encode_edge(ea_pad, w128, b128):
    blk = 2048

    def body(a_ref, w_ref, b_ref, o_ref):
        pid = pl.program_id(0)
        rows = lax.broadcasted_iota(jnp.int32, (blk, 1), 0) + pid * blk
        valid = rows < E
        ez = jnp.maximum(a_ref[...] * w_ref[...] + b_ref[...], 0.0)
        colio = lax.broadcasted_iota(jnp.int32, (blk, 128), 1)
        res = ez + jnp.where(colio == 64, 1.0, 0.0)
        o_ref[...] = jnp.where(valid, res, 0.0)

    return pl.pallas_call(
        body,
        grid=(EPAD // blk,),
        in_specs=[pl.BlockSpec((blk, 1), lambda i: (i, 0)),
                  _full((1, 128)), _full((1, 128))],
        out_specs=pl.BlockSpec((blk, 128), lambda i: (i, 0)),
        out_shape=jax.ShapeDtypeStruct((EPAD, 128), jnp.float32),
    )(ea_pad, w128, b128)


def _loope_div(parts, sel_e, sel_deg):
    blk = 2000

    def body(p_ref, se_ref, sd_ref, o_ref):
        s = p_ref[0] + p_ref[1]
        esum = _dot_hi(s, se_ref[...])
        deg = _dot_hi(s, sd_ref[...])
        o_ref[...] = esum / jnp.maximum(deg, 1.0)

    return pl.pallas_call(
        body,
        grid=(N // blk,),
        in_specs=[pl.BlockSpec((NC, blk, 128), lambda i: (0, i, 0)),
                  _full((128, EH)), _full((128, EH))],
        out_specs=pl.BlockSpec((blk, EH), lambda i: (i, 0)),
        out_shape=jax.ShapeDtypeStruct((N, EH), jnp.float32),
    )(parts, sel_e, sel_deg)


def _proj(h, loop_e, wl, bl, wr, br, we, attbd):
    blk = 2000

    def body(h_ref, le_ref, wl_ref, bl_ref, wr_ref, br_ref, we_ref, at_ref,
             xl_ref, xr_ref):
        h_b = h_ref[...]
        xl = _dot(h_b, wl_ref[...]) + bl_ref[...]
        xr = _dot(h_b, wr_ref[...]) + br_ref[...]
        ul = _dot(le_ref[...], we_ref[...])
        m = xl + xr + ul
        a = jnp.where(m > 0, m, 0.2 * m)
        c128 = _dot_hi(a, at_ref[...])
        xl_ref[...] = xl
        xr_ref[...] = jnp.concatenate([xr, c128], axis=1)

    return pl.pallas_call(
        body,
        grid=(N // blk,),
        in_specs=[pl.BlockSpec((blk, HID), lambda i: (i, 0)),
                  pl.BlockSpec((blk, EH), lambda i: (i, 0)),
                  _full((HID, HID)), _full((1, HID)),
                  _full((HID, HID)), _full((1, HID)),
                  _full((EH, HID)), _full((HID, 128))],
        out_specs=[pl.BlockSpec((blk, HID), lambda i: (i, 0)),
                   pl.BlockSpec((blk, HID + 128), lambda i: (i, 0))],
        out_shape=[jax.ShapeDtypeStruct((N, HID), jnp.float32),
                   jax.ShapeDtypeStruct((N, HID + 128), jnp.float32)],
    )(h, loop_e, wl, bl, wr, br, we, attbd)


def _edge_fused(xlg, xrg, e2p, we, attbd, expand, nheads):
    blk = 2048

    def body(xl_ref, xr_ref, e_ref, we_ref, at_ref, exp_ref,
             ex_ref, m0_ref, m1_ref):
        pid = pl.program_id(0)
        xr384 = xr_ref[...]
        xl = xl_ref[...]
        u = _dot(e_ref[...], we_ref[...])
        m = xl + xr384[:, :HID] + u
        a = jnp.where(m > 0, m, 0.2 * m)
        alpha = _dot_hi(a, at_ref[...])
        ex = jnp.exp(alpha - xr384[:, HID:])
        rows = lax.broadcasted_iota(jnp.int32, (blk, 1), 0) + pid * blk
        cols = lax.broadcasted_iota(jnp.int32, (blk, 128), 1)
        ex = jnp.where((rows < E2) & (cols < nheads), ex, 0.0)
        ex_ref[...] = ex
        alx = _dot_hi(ex, exp_ref[...])
        prod = xl * alx
        m0_ref[...] = prod[:, :128]
        m1_ref[...] = prod[:, 128:]

    return pl.pallas_call(
        body,
        grid=(E2PAD // blk,),
        in_specs=[pl.BlockSpec((blk, HID), lambda i: (i, 0)),
                  pl.BlockSpec((blk, HID + 128), lambda i: (i, 0)),
                  pl.BlockSpec((blk, EH), lambda i: (i, 0)),
                  _full((EH, HID)), _full((HID, 128)), _full((128, HID))],
        out_specs=[pl.BlockSpec((blk, 128), lambda i: (i, 0)),
                   pl.BlockSpec((blk, 128), lambda i: (i, 0)),
                   pl.BlockSpec((blk, 128), lambda i: (i, 0))],
        out_shape=[jax.ShapeDtypeStruct((E2PAD, 128), jnp.float32),
                   jax.ShapeDtypeStruct((E2PAD, 128), jnp.float32),
                   jax.ShapeDtypeStruct((E2PAD, 128), jnp.float32)],
    )(xlg, xrg, e2p, we, attbd, expand)


def _layer_out(p0, p1, den, expand, bias, hprev, residual):
    blk = 2000

    def body(p0_ref, p1_ref, dn_ref, exp_ref, b_ref, h_ref, o_ref):
        s0 = p0_ref[0] + p0_ref[1]
        s1 = p1_ref[0] + p1_ref[1]
        den = dn_ref[0] + dn_ref[1]
        denx = _dot_hi(den, exp_ref[...]) + 1e-16
        raw = jnp.concatenate([s0, s1], axis=1)
        out = raw / denx + b_ref[...]
        hn = _ln_relu(out)
        o_ref[...] = h_ref[...] + hn if residual else hn

    return pl.pallas_call(
        body,
        grid=(N // blk,),
        in_specs=[pl.BlockSpec((NC, blk, 128), lambda i: (0, i, 0)),
                  pl.BlockSpec((NC, blk, 128), lambda i: (0, i, 0)),
                  pl.BlockSpec((NC, blk, 128), lambda i: (0, i, 0)),
                  _full((128, HID)),
                  _full((1, HID)),
                  pl.BlockSpec((blk, HID), lambda i: (i, 0))],
        out_specs=pl.BlockSpec((blk, HID), lambda i: (i, 0)),
        out_shape=jax.ShapeDtypeStruct((N, HID), jnp.float32),
    )(p0, p1, den, expand, bias, hprev)


def _head(h, ps, dims):
    blk = 2000
    d0, d1, d2 = dims

    def body(h_ref, w0, b0, w1, b1, w2, b2, o_ref):
        z = _ln_relu(_dot(h_ref[...], w0[...]) + b0[...])
        z = jnp.maximum(_dot(z, w1[...]) + b1[...], 0.0)
        o_ref[...] = _dot(z, w2[...]) + b2[...]

    return pl.pallas_call(
        body,
        grid=(N // blk,),
        in_specs=[pl.BlockSpec((blk, HID), lambda i: (i, 0)),
                  _full((HID, d0)), _full((1, d0)),
                  _full((d0, d1)), _full((1, d1)),
                  _full((d1, d2)), _full((1, d2))],
        out_specs=pl.BlockSpec((blk, d2), lambda i: (i, 0)),
        out_shape=jax.ShapeDtypeStruct((N, d2), jnp.float32),
    )(h, ps[0]["W"], ps[0]["b"].reshape(1, -1),
      ps[1]["W"], ps[1]["b"].reshape(1, -1),
      ps[2]["W"], ps[2]["b"].reshape(1, -1))


def _pool(h, batch_col):
    blk = 2000

    def body(h_ref, b_ref, xsum_ref, xmax_ref, cnt_ref):
        @pl.when(pl.program_id(0) == 0)
        def _init():
            xsum_ref[...] = jnp.zeros_like(xsum_ref)
            xmax_ref[...] = jnp.full_like(xmax_ref, -jnp.inf)
            cnt_ref[...] = jnp.zeros_like(cnt_ref)

        hb = h_ref[...]
        bb = b_ref[...]
        biota = lax.broadcasted_iota(jnp.int32, (blk, B), 1)
        oh = (bb == biota).astype(jnp.float32)
        xsum_ref[...] += lax.dot_general(
            oh, hb, (((0,), (0,)), ((), ())),
            preferred_element_type=jnp.float32,
            precision=lax.Precision.HIGHEST)
        cnt_ref[...] += lax.dot_general(
            oh, jnp.ones((blk, 128), jnp.float32), (((0,), (0,)), ((), ())),
            preferred_element_type=jnp.float32,
            precision=lax.Precision.HIGHEST)
        for bseg in range(B):
            mb = bb == bseg
            mx = jnp.max(jnp.where(mb, hb, -jnp.inf), axis=0, keepdims=True)
            xmax_ref[bseg:bseg + 1, :] = jnp.maximum(
                xmax_ref[bseg:bseg + 1, :], mx)

    return pl.pallas_call(
        body,
        grid=(N // blk,),
        in_specs=[pl.BlockSpec((blk, HID), lambda i: (i, 0)),
                  pl.BlockSpec((blk, 1), lambda i: (i, 0))],
        out_specs=[_full((B, HID)), _full((B, HID)), _full((B, 128))],
        out_shape=[jax.ShapeDtypeStruct((B, HID), jnp.float32),
                   jax.ShapeDtypeStruct((B, HID), jnp.float32),
                   jax.ShapeDtypeStruct((B, 128), jnp.float32)],
    )(h, batch_col)


def _mk_head(xsum, xmax, cnt, pw_pad, pm):
    def body(xs_ref, xm_ref, cn_ref, pw_ref, w0, b0, w1, b1, w2, b2, w3, b3,
             o_ref):
        pw = pw_ref[...]
        mxw = jnp.max(pw, axis=-1, keepdims=True)
        ew = jnp.exp(pw - mxw)
        w = ew / jnp.sum(ew, axis=-1, keepdims=True)
        w0_, w1_, w2_ = w[:, 0:1], w[:, 1:2], w[:, 2:3]
        cnt1 = jnp.maximum(cn_ref[:, 0:1], 1.0)
        xsum = xs_ref[...]
        xmean = xsum / cnt1
        xmax = xm_ref[...]
        xmax = jnp.where(xmax > -3e38, xmax, 0.0)
        xp = jnp.concatenate([xmean * w0_, xmax * w1_, xsum * w2_], axis=1)
        z = _ln_relu(_dot(xp, w0[...]) + b0[...])
        z = _ln_relu(_dot(z, w1[...]) + b1[...])
        z = jnp.maximum(_dot(z, w2[...]) + b2[...], 0.0)
        o_ref[...] = _dot(z, w3[...]) + b3[...]

    args = [xsum, xmax, cnt, pw_pad]
    for p in pm:
        args += [p["W"], p["b"].reshape(1, -1)]
    return pl.pallas_call(
        body,
        out_shape=jax.ShapeDtypeStruct((B, 1), jnp.float32),
    )(*args)


def kernel(x, edge_index, edge_attr, batch, params):
    f32 = jnp.float32
    src, dst = edge_index[0], edge_index[1]
    ar = jnp.arange(N, dtype=jnp.int32)
    zpad2 = jnp.zeros((E2PAD - E2,), jnp.int32)
    src2p = jnp.concatenate([src, ar, zpad2])
    dst2p = jnp.concatenate([dst, ar, zpad2])
    dstp = jnp.concatenate([dst, jnp.zeros((EPAD - E,), jnp.int32)])

    x8 = jnp.pad(x, ((0, 0), (0, 5)))
    w8 = jnp.pad(params["ne"]["W"], ((0, 5), (0, 0)))
    h = _encode_node(x8, w8, params["ne"]["b"].reshape(1, -1))
    ea_pad = jnp.pad(edge_attr, ((0, EPAD - E), (0, 0)))
    w128 = jnp.pad(params["ee"]["W"], ((0, 0), (0, 64)))
    b128 = jnp.pad(params["ee"]["b"].reshape(1, -1), ((0, 0), (0, 64)))
    vals128 = _encode_edge(ea_pad, w128, b128)

    zeros_n128 = jnp.zeros((NPAD, 128), f32)
    le_parts = _sc_scatter(NPAD, 128, EPAD, "scat_loope", slots=4, chunk=64)(vals128, dstp,
                                                          zeros_n128)
    colio = jnp.arange(128)
    sel_e = (colio[:, None] == jnp.arange(EH)[None, :]).astype(f32)
    sel_deg = (colio[:, None] == 64).astype(f32) * jnp.ones((1, EH), f32)
    loop_e = _loope_div(le_parts, sel_e, sel_deg)

    e2p = jnp.concatenate(
        [lax.slice(vals128, (0, 0), (E, EH)), loop_e,
         jnp.zeros((E2PAD - E2, EH), f32)], axis=0)

    gat_h = _sc_gather(HID, E2PAD, "gather_hid", slots=3)
    gat_384 = _sc_gather(HID + 128, E2PAD, "gather_384", slots=2)
    scat_den = _sc_scatter(NPAD, 128, E2PAD, "scat_den", slots=4, chunk=64)
    scat_msg = _sc_scatter(NPAD, 128, E2PAD, "scat_msg", slots=4, chunk=64)

    idxh = jnp.arange(HID)
    for i in range(4):
        p = params["gat"][i]
        nh = 8 if i < 3 else 1
        ch = HID // nh
        attbd = jnp.zeros((HID, 128), f32).at[idxh, idxh // ch].set(
            p["att"].reshape(-1))
        expand = (jnp.arange(128)[:, None] == (idxh[None, :] // ch)).astype(f32)
        xl, xr384 = _proj(h, loop_e, p["Wl"], p["bl"].reshape(1, -1),
                          p["Wr"], p["br"].reshape(1, -1), p["We"], attbd)
        xlg = gat_h(xl, src2p)
        xrg = gat_384(xr384, dst2p)
        ex, msg0, msg1 = _edge_fused(xlg, xrg, e2p, p["We"], attbd, expand, nh)
        den_parts = scat_den(ex, dst2p, zeros_n128)
        p0 = scat_msg(msg0, dst2p, zeros_n128)
        p1 = scat_msg(msg1, dst2p, zeros_n128)
        h = _layer_out(p0, p1, den_parts, expand, p["b"].reshape(1, -1), h,
                       residual=(i > 0))

    proc = _head(h, params["proc"], (HID, HID // 2, NPROC))
    st = _head(h, params["st"], (HID // 2, HID // 4, 1))
    et = _head(h, params["et"], (HID // 2, HID // 4, 1))

    xsum, xmax, cnt = _pool(h, batch.reshape(-1, 1))
    pw_pad = jnp.concatenate(
        [params["pw"], jnp.full((125,), -jnp.inf, f32)]).reshape(1, 128)
    mk = _mk_head(xsum, xmax, cnt, pw_pad, params["mk"])
    return (proc, st, et, mk)
